# baseline (speedup 1.0000x reference)
import jax
import jax.numpy as jnp
from jax import lax
from jax.experimental import pallas as pl
from jax.experimental.pallas import tpu as pltpu
from jax.experimental.pallas import tpu_sc as plsc

N_NODES = 10000
N_EDGES = 320000
D_FEAT = 128
D_AGG = D_FEAT + 3
N_CORES = 2
N_SUBCORES = 16
NW = N_CORES * N_SUBCORES
LANES = 16
CHUNK = 128
N_CHUNKS = N_EDGES // CHUNK
BASE_CHUNKS = N_CHUNKS // NW


def _gather_body(feat_hbm, pos_hbm, ei_hbm, agg_hbm, geo_hbm,
                 pos_v, s0, s1, d0, d1, f0, f1, a0, a1, g0, g1,
                 isem0, isem1, gsem0, gsem1, wsem0, wsem1):
    wid = lax.axis_index("s") * N_CORES + lax.axis_index("c")
    n_mine = BASE_CHUNKS + 2 * (wid < 2).astype(jnp.int32)
    start_chunk = BASE_CHUNKS * wid + 2 * jnp.minimum(wid, 2)
    n2 = n_mine // 2

    sidx, didx = (s0, s1), (d0, d1)
    fbuf, abuf, gbuf = (f0, f1), (a0, a1), (g0, g1)
    isem, gsem, wsem = (isem0, isem1), (gsem0, gsem1), (wsem0, wsem1)

    pltpu.sync_copy(pos_hbm, pos_v)
    e0g = start_chunk * CHUNK
    pltpu.sync_copy(ei_hbm.at[0, pl.ds(e0g, CHUNK)], s0)
    pltpu.sync_copy(ei_hbm.at[1, pl.ds(e0g, CHUNK)], d0)
    pltpu.async_copy(ei_hbm.at[0, pl.ds(e0g + CHUNK, CHUNK)], s1, isem1)
    pltpu.async_copy(ei_hbm.at[1, pl.ds(e0g + CHUNK, CHUNK)], d1, isem1)
    pltpu.async_copy(feat_hbm.at[s0], f0, gsem0)

    lane = lax.iota(jnp.int32, LANES)

    def pair(kk, carry):
        for b in (0, 1):
            k = 2 * kk + b
            goff = (start_chunk + k) * CHUNK
            fb, ab, gb = fbuf[b], abuf[b], gbuf[b]

            pltpu.make_async_copy(feat_hbm.at[sidx[b]], fb, gsem[b]).wait()

            def issue_next():
                pltpu.make_async_copy(
                    ei_hbm.at[0, pl.ds(0, CHUNK)], sidx[1 - b],
                    isem[1 - b]).wait()
                pltpu.make_async_copy(
                    ei_hbm.at[1, pl.ds(0, CHUNK)], didx[1 - b],
                    isem[1 - b]).wait()
                pltpu.async_copy(
                    feat_hbm.at[sidx[1 - b]], fbuf[1 - b], gsem[1 - b])
            if b == 0:
                issue_next()
            else:
                pl.when(kk < n2 - 1)(issue_next)

            @pl.when(kk >= 1)
            def _():
                pltpu.make_async_copy(
                    ab, agg_hbm.at[:, pl.ds(goff, CHUNK)], wsem[b]).wait()
                pltpu.make_async_copy(
                    gb, geo_hbm.at[:, pl.ds(goff, CHUNK)], wsem[b]).wait()

            for g in range(CHUNK // LANES):
                eid = lane + g * LANES
                si3 = sidx[b][pl.ds(g * LANES, LANES)] * 3
                di3 = didx[b][pl.ds(g * LANES, LANES)] * 3
                for j in range(3):
                    rj = jnp.full((LANES,), j, jnp.int32)
                    ps = plsc.load_gather(pos_v, [si3 + j])
                    pd = plsc.load_gather(pos_v, [di3 + j])
                    plsc.store_scatter(ab, [rj, eid], ps - pd)
                    plsc.store_scatter(gb, [rj, eid], ps)
                    plsc.store_scatter(gb, [rj + 3, eid], pd)

            def shift(eb, carry2):
                e0 = eb * LANES
                ln = lax.iota(jnp.int32, LANES)
                xidx = {s: ln ^ s for s in (1, 2, 4, 8)}
                xmask = {s: (ln & s) != 0 for s in (1, 2, 4, 8)}
                for jb in range(D_FEAT // LANES):
                    v = [fb[e0 + i, pl.ds(jb * LANES, LANES)]
                         for i in range(LANES)]
                    for s in (1, 2, 4, 8):
                        nv = list(v)
                        for i in range(LANES):
                            if i & s:
                                continue
                            a, b = v[i], v[i | s]
                            bp = b.at[xidx[s]].get(mode="promise_in_bounds")
                            ap = a.at[xidx[s]].get(mode="promise_in_bounds")
                            nv[i] = jnp.where(xmask[s], bp, a)
                            nv[i | s] = jnp.where(xmask[s], b, ap)
                        v = nv
                    for i in range(LANES):
                        ab[3 + jb * LANES + i, pl.ds(e0, LANES)] = v[i]
                return carry2

            lax.fori_loop(0, CHUNK // LANES, shift, 0)

            pltpu.async_copy(ab, agg_hbm.at[:, pl.ds(goff, CHUNK)], wsem[b])
            pltpu.async_copy(gb, geo_hbm.at[:, pl.ds(goff, CHUNK)], wsem[b])

            @pl.when(kk < n2 - 1)
            def _():
                pltpu.async_copy(
                    ei_hbm.at[0, pl.ds((start_chunk + k + 2) * CHUNK, CHUNK)],
                    sidx[b], isem[b])
                pltpu.async_copy(
                    ei_hbm.at[1, pl.ds((start_chunk + k + 2) * CHUNK, CHUNK)],
                    didx[b], isem[b])
        return carry

    lax.fori_loop(0, n2, pair, 0)

    for b in (0, 1):
        pltpu.make_async_copy(
            abuf[b], agg_hbm.at[:, pl.ds(0, CHUNK)], wsem[b]).wait()
        pltpu.make_async_copy(
            gbuf[b], geo_hbm.at[:, pl.ds(0, CHUNK)], wsem[b]).wait()


_gather = pl.kernel(
    _gather_body,
    out_type=(
        jax.ShapeDtypeStruct((D_AGG, N_EDGES), jnp.float32),
        jax.ShapeDtypeStruct((6, N_EDGES), jnp.float32),
    ),
    mesh=plsc.VectorSubcoreMesh(
        core_axis_name="c", subcore_axis_name="s",
        num_cores=N_CORES, num_subcores=N_SUBCORES),
    compiler_params=pltpu.CompilerParams(
        needs_layout_passes=False, use_tc_tiling_on_sc=True),
    scratch_types=[
        pltpu.VMEM((N_NODES * 3,), jnp.float32),
        pltpu.VMEM((CHUNK,), jnp.int32),
        pltpu.VMEM((CHUNK,), jnp.int32),
        pltpu.VMEM((CHUNK,), jnp.int32),
        pltpu.VMEM((CHUNK,), jnp.int32),
        pltpu.VMEM((CHUNK, D_FEAT), jnp.float32),
        pltpu.VMEM((CHUNK, D_FEAT), jnp.float32),
        pltpu.VMEM((D_AGG, CHUNK), jnp.float32),
        pltpu.VMEM((D_AGG, CHUNK), jnp.float32),
        pltpu.VMEM((6, CHUNK), jnp.float32),
        pltpu.VMEM((6, CHUNK), jnp.float32),
        pltpu.SemaphoreType.DMA,
        pltpu.SemaphoreType.DMA,
        pltpu.SemaphoreType.DMA,
        pltpu.SemaphoreType.DMA,
        pltpu.SemaphoreType.DMA,
        pltpu.SemaphoreType.DMA,
    ],
)


def kernel(pos, feat, edge_index):
    ei = edge_index.astype(jnp.int32)
    agg_t, geo_t = _gather(feat, pos.reshape(N_NODES * 3), ei)
    return agg_t.T, geo_t.T
